# baseline (speedup 1.0000x reference)
import functools

import jax
import jax.numpy as jnp
from jax import lax
from jax.experimental import pallas as pl
from jax.experimental.pallas import tpu as pltpu
from jax.experimental.pallas import tpu_sc as plsc

_NC, _NS, _L = 2, 16, 16
_NW = _NC * _NS
_CP = 256
_NP = 2


@functools.lru_cache(maxsize=None)
def _make_sc(B, S, V):
    cols_per_w = B // _NW
    slots = cols_per_w * _CP
    n_chunks = -(-S // _L)
    half = V // _NP

    mesh = plsc.VectorSubcoreMesh(
        core_axis_name="c", subcore_axis_name="s",
        num_cores=_NC, num_subcores=_NS)

    @functools.partial(
        pl.kernel,
        out_type=jax.ShapeDtypeStruct((B,), jnp.float32),
        mesh=mesh,
        scratch_types=[
            pltpu.VMEM((slots,), jnp.int32),
            pltpu.VMEM((half,), jnp.float32),
            pltpu.VMEM((half,), jnp.int32),
            pltpu.VMEM((cols_per_w,), jnp.float32),
        ],
        compiler_params=pltpu.CompilerParams(needs_layout_passes=False),
    )
    def sc(text_hbm, w_hbm, out_hbm, tok_v, wch_v, tag_v, out_v):
        wid = lax.axis_index("s") * _NC + lax.axis_index("c")
        pltpu.sync_copy(text_hbm.at[wid], tok_v)

        lane = lax.iota(jnp.int32, _L)
        out0 = jnp.zeros((_L,), jnp.float32)
        out1 = jnp.zeros((_L,), jnp.float32)
        for p in range(_NP):
            lo = p * half
            pltpu.sync_copy(w_hbm.at[pl.ds(lo, half)], wch_v)

            def col_body(col, outs, lo=lo):
                out0, out1 = outs
                base = col * _CP
                tvecs = []
                masks = []
                for c in range(n_chunks):
                    idx = tok_v[pl.ds(base + c * _L, _L)]
                    t = idx - lo
                    pos = lane + c * _L
                    if p == 0:
                        inr = idx < half
                    else:
                        inr = idx >= lo
                    if (c + 1) * _L > S:
                        inr = jnp.logical_and(inr, pos < S)
                    tvecs.append(t)
                    masks.append(inr)
                    plsc.store_scatter(tag_v, [t], pos, mask=inr)
                acc = jnp.zeros((_L,), jnp.float32)
                for c in range(n_chunks):
                    t, inr = tvecs[c], masks[c]
                    pos = lane + c * _L
                    tags = plsc.load_gather(tag_v, [t], mask=inr)
                    sel = jnp.logical_and(inr, tags == pos)
                    wv = plsc.load_gather(wch_v, [t], mask=sel)
                    acc = acc + jnp.where(sel, wv, jnp.float32(0))
                s = jnp.sum(acc)
                out0 = jnp.where(lane == col, out0 + s, out0)
                out1 = jnp.where(lane == col - _L, out1 + s, out1)
                return out0, out1

            out0, out1 = lax.fori_loop(0, cols_per_w, col_body, (out0, out1))

        out_v[pl.ds(0, _L)] = out0
        out_v[pl.ds(_L, _L)] = out1
        pltpu.sync_copy(out_v, out_hbm.at[pl.ds(wid * cols_per_w, cols_per_w)])

    return sc


def kernel(text, W, b):
    S, B = text.shape
    V = W.shape[1]
    t = jnp.pad(text.T.astype(jnp.int32), ((0, 0), (0, _CP - S)))
    t2 = t.reshape(_NW, (B // _NW) * _CP)
    out = _make_sc(B, S, V)(t2, W.reshape(-1))
    return out.reshape(B, 1) + b
